# baseline (speedup 1.0000x reference)
import jax
import jax.numpy as jnp
from jax import lax
from jax.experimental import pallas as pl
from jax.experimental.pallas import tpu as pltpu
from jax.experimental.pallas import tpu_sc as plsc

N = 10000
E = 320000
D = 128

NC = 2
NS = 16
NW = NC * NS

EPAD = 327680
E_PER_W = EPAD // NW
CHUNK = 80
N_CHUNKS = E_PER_W // CHUNK
NPAD = 10240
ROWS_PER_TILE = NPAD // NS
LANES = 16
DL = D // LANES


def _sc_scatter(state_hbm, w_hbm, row_hbm, col_hbm, zeros_hbm,
                out_hbm, deg_hbm,
                accum, degacc,
                idx_row0, idx_col0, wbuf0, idx_row1, idx_col1, wbuf1,
                wexp, rows_a0, rows_a1,
                sem_i0, sem_i1, sem_g0, sem_g1):
    c = lax.axis_index("c")
    s = lax.axis_index("s")
    wid = s * NC + c
    base0 = wid * E_PER_W

    zsl = pl.ds(s * ROWS_PER_TILE, ROWS_PER_TILE)
    pltpu.sync_copy(zeros_hbm.at[zsl], accum.at[zsl])

    def dzero(k, _):
        degacc[pl.ds(k * LANES, LANES)] = jnp.zeros((LANES,), jnp.float32)
        return _
    lax.fori_loop(0, NPAD // LANES, dzero, None)
    plsc.subcore_barrier()

    def issue_idx(g, ir, ic, wb, sem):
        base = base0 + g * CHUNK
        pltpu.async_copy(row_hbm.at[pl.ds(base, CHUNK)], ir, sem)
        pltpu.async_copy(col_hbm.at[pl.ds(base, CHUNK)], ic, sem)
        pltpu.async_copy(w_hbm.at[pl.ds(base, CHUNK)], wb, sem)

    def wait_idx(ir, ic, wb, sem):
        z = pl.ds(0, CHUNK)
        pltpu.make_async_copy(row_hbm.at[z], ir, sem).wait()
        pltpu.make_async_copy(col_hbm.at[z], ic, sem).wait()
        pltpu.make_async_copy(w_hbm.at[z], wb, sem).wait()

    def halfstep(g, ir_cur, ic_cur, wb_cur, ra_cur, sem_g_cur, sem_i_cur,
                 ir_nxt, ic_nxt, wb_nxt, ra_nxt, sem_g_nxt, sem_i_nxt):
        @pl.when(g + 1 < N_CHUNKS)
        def _():
            wait_idx(ir_nxt, ic_nxt, wb_nxt, sem_i_nxt)
            pltpu.async_copy(state_hbm.at[ic_nxt], ra_nxt, sem_g_nxt)

        def wexpand(k, _):
            kl = k * LANES
            wv = wb_cur[pl.ds(kl, LANES)]
            iv = ir_cur[pl.ds(kl, LANES)]
            plsc.addupdate_scatter(degacc, [iv], wv)
            for e16 in range(LANES):
                wexp[kl + e16, :] = jnp.full((LANES,), wv[e16], jnp.float32)
            return _
        lax.fori_loop(0, CHUNK // LANES, wexpand, None)

        pltpu.make_async_copy(state_hbm.at[ic_cur], ra_cur, sem_g_cur).wait()

        def edge(e, _):
            wv = wexp[e, :]
            for j in range(DL):
                sl = pl.ds(j * LANES, LANES)
                ra_cur[e, sl] = wv * ra_cur[e, sl]
            return _
        lax.fori_loop(0, CHUNK, edge, None)
        pltpu.sync_copy(ra_cur, accum.at[ir_cur], add=True)

        @pl.when(g + 2 < N_CHUNKS)
        def _():
            issue_idx(g + 2, ir_cur, ic_cur, wb_cur, sem_i_cur)

    issue_idx(0, idx_row0, idx_col0, wbuf0, sem_i0)
    issue_idx(1, idx_row1, idx_col1, wbuf1, sem_i1)
    wait_idx(idx_row0, idx_col0, wbuf0, sem_i0)
    pltpu.async_copy(state_hbm.at[idx_col0], rows_a0, sem_g0)

    def pair(i, _):
        g0 = 2 * i
        halfstep(g0, idx_row0, idx_col0, wbuf0, rows_a0, sem_g0, sem_i0,
                 idx_row1, idx_col1, wbuf1, rows_a1, sem_g1, sem_i1)
        halfstep(g0 + 1, idx_row1, idx_col1, wbuf1, rows_a1, sem_g1, sem_i1,
                 idx_row0, idx_col0, wbuf0, rows_a0, sem_g0, sem_i0)
        return _
    lax.fori_loop(0, N_CHUNKS // 2, pair, None)

    plsc.subcore_barrier()
    sl = pl.ds(s * ROWS_PER_TILE, ROWS_PER_TILE)
    pltpu.sync_copy(accum.at[sl], out_hbm.at[c, sl])
    pltpu.sync_copy(degacc, deg_hbm.at[wid])


def _tc_combine(p_ref, deg_ref, state_ref, o_ref):
    deg = jnp.sum(deg_ref[...], axis=0)
    o_ref[...] = p_ref[0] + p_ref[1] - deg[:, None] * state_ref[...]


@jax.jit
def kernel(state_variable, weights, edge_index):
    npad_e = EPAD - E
    row = jnp.concatenate([edge_index[0], jnp.full((npad_e,), N, jnp.int32)])
    col = jnp.concatenate([edge_index[1], jnp.zeros((npad_e,), jnp.int32)])
    weights = jnp.concatenate([weights, jnp.zeros((npad_e,), jnp.float32)])
    mesh = plsc.VectorSubcoreMesh(core_axis_name="c", subcore_axis_name="s")
    partial, degs = pl.kernel(
        _sc_scatter,
        mesh=mesh,
        compiler_params=pltpu.CompilerParams(needs_layout_passes=False),
        out_type=(
            jax.ShapeDtypeStruct((NC, NPAD, D), jnp.float32),
            jax.ShapeDtypeStruct((NW, NPAD), jnp.float32),
        ),
        scratch_types=[
            pltpu.VMEM_SHARED((NPAD, D), jnp.float32),
            pltpu.VMEM((NPAD,), jnp.float32),
            pltpu.VMEM((CHUNK,), jnp.int32),
            pltpu.VMEM((CHUNK,), jnp.int32),
            pltpu.VMEM((CHUNK,), jnp.float32),
            pltpu.VMEM((CHUNK,), jnp.int32),
            pltpu.VMEM((CHUNK,), jnp.int32),
            pltpu.VMEM((CHUNK,), jnp.float32),
            pltpu.VMEM((CHUNK, LANES), jnp.float32),
            pltpu.VMEM((CHUNK, D), jnp.float32),
            pltpu.VMEM((CHUNK, D), jnp.float32),
            pltpu.SemaphoreType.DMA,
            pltpu.SemaphoreType.DMA,
            pltpu.SemaphoreType.DMA,
            pltpu.SemaphoreType.DMA,
        ],
    )(state_variable, weights, row, col, jnp.zeros((NPAD, D), jnp.float32))

    nblk = 10
    blk = NPAD // nblk
    return pl.pallas_call(
        _tc_combine,
        grid=(nblk,),
        in_specs=[
            pl.BlockSpec((NC, blk, D), lambda i: (0, i, 0)),
            pl.BlockSpec((NW, blk), lambda i: (0, i)),
            pl.BlockSpec((blk, D), lambda i: (i, 0)),
        ],
        out_specs=pl.BlockSpec((blk, D), lambda i: (i, 0)),
        out_shape=jax.ShapeDtypeStruct((N, D), jnp.float32),
    )(partial, degs, state_variable)


# speedup vs baseline: 1.0131x; 1.0131x over previous
import jax
import jax.numpy as jnp
from jax import lax
from jax.experimental import pallas as pl
from jax.experimental.pallas import tpu as pltpu
from jax.experimental.pallas import tpu_sc as plsc

N = 10000
E = 320000
D = 128

NC = 2
NS = 16
NW = NC * NS

EPAD = 327680
E_PER_W = EPAD // NW
CHUNK = 64
N_CHUNKS = E_PER_W // CHUNK
NPAD = 10240
ROWS_PER_TILE = NPAD // NS
LANES = 16
DL = D // LANES

NBUF = 4
NIDX = 2 * NBUF


def _sc_scatter(state_hbm, w_hbm, row_hbm, col_hbm, zeros_hbm, zerosd_hbm,
                out_hbm, deg_hbm,
                accum, degsh,
                irs, ics, wbs, ras, wexp,
                sem_is, sem_gs):
    c = lax.axis_index("c")
    s = lax.axis_index("s")
    wid = s * NC + c
    base0 = wid * E_PER_W

    zsl = pl.ds(s * ROWS_PER_TILE, ROWS_PER_TILE)
    pltpu.sync_copy(zeros_hbm.at[zsl], accum.at[zsl])

    @pl.when(s == 0)
    def _():
        pltpu.sync_copy(zerosd_hbm, degsh)
    plsc.subcore_barrier()

    def issue_idx(g, j):
        base = base0 + g * CHUNK
        pltpu.async_copy(row_hbm.at[pl.ds(base, CHUNK)], irs[j], sem_is[j])
        pltpu.async_copy(col_hbm.at[pl.ds(base, CHUNK)], ics[j], sem_is[j])
        pltpu.async_copy(w_hbm.at[pl.ds(base, CHUNK)], wbs[j], sem_is[j])

    def wait_idx(j):
        z = pl.ds(0, CHUNK)
        pltpu.make_async_copy(row_hbm.at[z], irs[j], sem_is[j]).wait()
        pltpu.make_async_copy(col_hbm.at[z], ics[j], sem_is[j]).wait()
        pltpu.make_async_copy(w_hbm.at[z], wbs[j], sem_is[j]).wait()

    for j in range(NIDX):
        issue_idx(j, j)
    for b in range(NBUF):
        wait_idx(b)
        pltpu.async_copy(state_hbm.at[ics[b]], ras[b], sem_gs[b])

    def group(i2, _):
        h0 = i2 * NIDX
        for v in range(NIDX):
            h = h0 + v
            jr = v % NBUF
            ji = v
            pv = (v + NBUF - 1) % NIDX
            pr = pv % NBUF
            pre = h + NBUF - 1

            @pl.when(jnp.logical_and(pre >= NBUF, pre < N_CHUNKS))
            def _():
                wait_idx(pv)
                pltpu.async_copy(state_hbm.at[ics[pv]], ras[pr], sem_gs[pr])

            pltpu.sync_copy(wbs[ji], degsh.at[irs[ji]], add=True)

            def wexpand(k, _):
                kl = k * LANES
                wv = wbs[ji][pl.ds(kl, LANES)]
                for e16 in range(LANES):
                    wexp[kl + e16, :] = jnp.full((LANES,), wv[e16], jnp.float32)
                return _
            lax.fori_loop(0, CHUNK // LANES, wexpand, None)

            pltpu.make_async_copy(state_hbm.at[ics[ji]], ras[jr], sem_gs[jr]).wait()

            def edge(e, _):
                wv = wexp[e, :]
                for j in range(DL):
                    sl = pl.ds(j * LANES, LANES)
                    ras[jr][e, sl] = wv * ras[jr][e, sl]
                return _
            lax.fori_loop(0, CHUNK, edge, None)
            pltpu.sync_copy(ras[jr], accum.at[irs[ji]], add=True)

            @pl.when(h + NIDX < N_CHUNKS)
            def _():
                issue_idx(h + NIDX, ji)
        return _
    lax.fori_loop(0, N_CHUNKS // NIDX, group, None)

    plsc.subcore_barrier()
    sl = pl.ds(s * ROWS_PER_TILE, ROWS_PER_TILE)
    pltpu.sync_copy(accum.at[sl], out_hbm.at[c, sl])

    @pl.when(s == 0)
    def _():
        pltpu.sync_copy(degsh, deg_hbm.at[c])


def _tc_combine(p_ref, deg_ref, state_ref, o_ref):
    deg = jnp.sum(deg_ref[...], axis=0)
    o_ref[...] = p_ref[0] + p_ref[1] - deg[:, None] * state_ref[...]


@jax.jit
def kernel(state_variable, weights, edge_index):
    npad_e = EPAD - E
    row = jnp.concatenate([edge_index[0], jnp.full((npad_e,), N, jnp.int32)])
    col = jnp.concatenate([edge_index[1], jnp.zeros((npad_e,), jnp.int32)])
    weights = jnp.concatenate([weights, jnp.zeros((npad_e,), jnp.float32)])
    mesh = plsc.VectorSubcoreMesh(core_axis_name="c", subcore_axis_name="s")
    partial, degs = pl.kernel(
        _sc_scatter,
        mesh=mesh,
        compiler_params=pltpu.CompilerParams(needs_layout_passes=False),
        out_type=(
            jax.ShapeDtypeStruct((NC, NPAD, D), jnp.float32),
            jax.ShapeDtypeStruct((NC, NPAD), jnp.float32),
        ),
        scratch_types=[
            pltpu.VMEM_SHARED((NPAD, D), jnp.float32),
            pltpu.VMEM_SHARED((NPAD,), jnp.float32),
            [pltpu.VMEM((CHUNK,), jnp.int32) for _ in range(NIDX)],
            [pltpu.VMEM((CHUNK,), jnp.int32) for _ in range(NIDX)],
            [pltpu.VMEM((CHUNK,), jnp.float32) for _ in range(NIDX)],
            [pltpu.VMEM((CHUNK, D), jnp.float32) for _ in range(NBUF)],
            pltpu.VMEM((CHUNK, LANES), jnp.float32),
            [pltpu.SemaphoreType.DMA for _ in range(NIDX)],
            [pltpu.SemaphoreType.DMA for _ in range(NBUF)],
        ],
    )(state_variable, weights, row, col,
      jnp.zeros((NPAD, D), jnp.float32), jnp.zeros((NPAD,), jnp.float32))

    nblk = 10
    blk = NPAD // nblk
    return pl.pallas_call(
        _tc_combine,
        grid=(nblk,),
        in_specs=[
            pl.BlockSpec((NC, blk, D), lambda i: (0, i, 0)),
            pl.BlockSpec((NC, blk), lambda i: (0, i)),
            pl.BlockSpec((blk, D), lambda i: (i, 0)),
        ],
        out_specs=pl.BlockSpec((blk, D), lambda i: (i, 0)),
        out_shape=jax.ShapeDtypeStruct((N, D), jnp.float32),
    )(partial, degs, state_variable)
